# Initial kernel scaffold; baseline (speedup 1.0000x reference)
import jax
import jax.numpy as jnp
from jax.experimental import pallas as pl


def kernel(x, expert_latents, W1, W2, Wq, sub_keys):
    raise NotImplementedError("write your pallas kernel here")


# speedup vs baseline: 16.7762x; 16.7762x over previous
import functools

import jax
import jax.numpy as jnp
from jax import lax
from jax.experimental import pallas as pl
from jax.experimental.pallas import tpu as pltpu

_B, _S, _D = 2, 2048, 2048
_E, _K, _H = 64, 2, 4
_DL, _DH = 64, 128
_NSUB, _DHALF = 8, 64
_N = _B * _S
_BLK = 512
_NBLK = _N // _BLK

_NEG_INF = float("-inf")


def _top2(s, width):
    iota = lax.broadcasted_iota(jnp.int32, s.shape, 1)
    v0 = jnp.max(s, axis=1, keepdims=True)
    i0 = jnp.min(jnp.where(s >= v0, iota, width), axis=1, keepdims=True)
    sm = jnp.where(iota == i0, _NEG_INF, s)
    v1 = jnp.max(sm, axis=1, keepdims=True)
    i1 = jnp.min(jnp.where(sm >= v1, iota, width), axis=1, keepdims=True)
    return v0, i0, v1, i1


def _moe_body(x_ref, wq_ref, sk1_ref, sk2_ref, el_ref, w1_ref, wu_ref, wv_ref,
              o_ref):
    xb = x_ref[...]
    q = jnp.dot(xb, wq_ref[...], preferred_element_type=jnp.float32)
    x_proj = jnp.dot(xb, wu_ref[...], preferred_element_type=jnp.float32)

    h_all = jnp.dot(el_ref[...], w1_ref[...], preferred_element_type=jnp.float32)
    h_all = h_all * jax.nn.sigmoid(1.702 * h_all)

    dot_all = lax.dot_general(x_proj, h_all, (((1,), (1,)), ((), ())),
                              preferred_element_type=jnp.float32)

    s1_all = jnp.dot(q[:, :_H * _DHALF], sk1_ref[...],
                     preferred_element_type=jnp.float32)
    s2_all = jnp.dot(q[:, _H * _DHALF:], sk2_ref[...],
                     preferred_element_type=jnp.float32)

    iota_e = lax.broadcasted_iota(jnp.int32, (_BLK, _E), 1)
    w_acc = jnp.zeros((_BLK, _E), jnp.float32)
    for h in range(_H):
        s1 = s1_all[:, h * _NSUB:(h + 1) * _NSUB]
        s2 = s2_all[:, h * _NSUB:(h + 1) * _NSUB]
        v1a, i1a, v1b, i1b = _top2(s1, _NSUB)
        v2a, i2a, v2b, i2b = _top2(s2, _NSUB)
        comb = jnp.concatenate(
            [v1a + v2a, v1a + v2b, v1b + v2a, v1b + v2b], axis=1)
        cidx = jnp.concatenate(
            [i1a * _NSUB + i2a, i1a * _NSUB + i2b,
             i1b * _NSUB + i2a, i1b * _NSUB + i2b], axis=1)
        sc0, p0, sc1, p1 = _top2(comb, _K * _K)
        iota4 = lax.broadcasted_iota(jnp.int32, (_BLK, _K * _K), 1)
        e0 = jnp.sum(jnp.where(iota4 == p0, cidx, 0), axis=1, keepdims=True)
        e1 = jnp.sum(jnp.where(iota4 == p1, cidx, 0), axis=1, keepdims=True)
        ex = jnp.exp(sc1 - sc0)
        denom = 1.0 + ex
        sw0 = 1.0 / denom
        sw1 = ex / denom
        m0 = iota_e == e0
        m1 = iota_e == e1
        d0 = jnp.sum(jnp.where(m0, dot_all, 0.0), axis=1, keepdims=True)
        d1 = jnp.sum(jnp.where(m1, dot_all, 0.0), axis=1, keepdims=True)
        act0 = d0 * jax.nn.sigmoid(1.702 * d0) * sw0
        act1 = d1 * jax.nn.sigmoid(1.702 * d1) * sw1
        w_acc = w_acc + jnp.where(m0, act0, 0.0) + jnp.where(m1, act1, 0.0)

    c = jnp.dot(w_acc, h_all, preferred_element_type=jnp.float32)
    o_ref[...] = jnp.dot(c, wv_ref[...],
                         preferred_element_type=jnp.float32) * (1.0 / _H)


@functools.partial(jax.jit, static_argnames=())
def _run(xf, wqr, sk1, sk2, el, w1, wu, wv):
    return pl.pallas_call(
        _moe_body,
        grid=(_NBLK,),
        in_specs=[
            pl.BlockSpec((_BLK, _D), lambda i: (i, 0)),
            pl.BlockSpec((_D, 2 * _H * _DHALF), lambda i: (0, 0)),
            pl.BlockSpec((_H * _DHALF, _H * _NSUB), lambda i: (0, 0)),
            pl.BlockSpec((_H * _DHALF, _H * _NSUB), lambda i: (0, 0)),
            pl.BlockSpec((_E, _DL), lambda i: (0, 0)),
            pl.BlockSpec((_DL, _DH), lambda i: (0, 0)),
            pl.BlockSpec((_D, _DH), lambda i: (0, 0)),
            pl.BlockSpec((_DH, _D), lambda i: (0, 0)),
        ],
        out_specs=pl.BlockSpec((_BLK, _D), lambda i: (i, 0)),
        out_shape=jax.ShapeDtypeStruct((_N, _D), jnp.float32),
        compiler_params=pltpu.CompilerParams(
            dimension_semantics=("arbitrary",)),
    )(xf, wqr, sk1, sk2, el, w1, wu, wv)


def kernel(x, expert_latents, W1, W2, Wq, sub_keys):
    xf = x.reshape(_N, _D)
    wqr = Wq.reshape(_D, _H, 2, _DHALF).transpose(0, 2, 1, 3).reshape(
        _D, 2 * _H * _DHALF)
    sk1 = jax.scipy.linalg.block_diag(
        *[sub_keys[0, h].T for h in range(_H)])
    sk2 = jax.scipy.linalg.block_diag(
        *[sub_keys[1, h].T for h in range(_H)])
    wu = W2[:, :_D].T
    wv = W2[:, _D:]
    out = _run(xf, wqr, sk1, sk2, expert_latents, W1, wu, wv)
    return out.reshape(_B, _S, _D)


# speedup vs baseline: 40.9255x; 2.4395x over previous
import functools

import jax
import jax.numpy as jnp
from jax import lax
from jax.experimental import pallas as pl
from jax.experimental.pallas import tpu as pltpu

_B, _S, _D = 2, 2048, 2048
_E, _K, _H = 64, 2, 4
_DL, _DH = 64, 128
_NSUB, _DHALF = 8, 64
_N = _B * _S
_BLK = 512
_NBLK = _N // _BLK

_NEG_INF = float("-inf")


def _top2_ax1(s, width):
    iota = lax.broadcasted_iota(jnp.int32, s.shape, 1)
    v0 = jnp.max(s, axis=1, keepdims=True)
    i0 = jnp.min(jnp.where(s >= v0, iota, width), axis=1, keepdims=True)
    sm = jnp.where(iota == i0, _NEG_INF, s)
    v1 = jnp.max(sm, axis=1, keepdims=True)
    i1 = jnp.min(jnp.where(sm >= v1, iota, width), axis=1, keepdims=True)
    return v0, i0, v1, i1


def _moe_body(x_ref, wq_ref, sk1_ref, sk2_ref, el_ref, w1_ref, wu_ref, wv_ref,
              o_ref):
    xb = x_ref[...]
    q = jnp.dot(xb, wq_ref[...], preferred_element_type=jnp.float32)
    x_proj = jnp.dot(xb, wu_ref[...], preferred_element_type=jnp.float32)

    h_all = jnp.dot(el_ref[...], w1_ref[...], preferred_element_type=jnp.float32)
    h_all = h_all * jax.nn.sigmoid(1.702 * h_all)

    dot_all = lax.dot_general(x_proj, h_all, (((1,), (1,)), ((), ())),
                              preferred_element_type=jnp.float32)
    dot_all_t = jnp.transpose(dot_all)

    s1_all = jnp.dot(q[:, :_H * _DHALF], sk1_ref[...],
                     preferred_element_type=jnp.float32)
    s2_all = jnp.dot(q[:, _H * _DHALF:], sk2_ref[...],
                     preferred_element_type=jnp.float32)
    s1t = jnp.transpose(s1_all).reshape(_H, _NSUB, _BLK)
    s2t = jnp.transpose(s2_all).reshape(_H, _NSUB, _BLK)

    v1a, i1a, v1b, i1b = _top2_ax1(s1t, _NSUB)
    v2a, i2a, v2b, i2b = _top2_ax1(s2t, _NSUB)
    comb = jnp.concatenate(
        [v1a + v2a, v1a + v2b, v1b + v2a, v1b + v2b], axis=1)
    cidx = jnp.concatenate(
        [i1a * _NSUB + i2a, i1a * _NSUB + i2b,
         i1b * _NSUB + i2a, i1b * _NSUB + i2b], axis=1)
    sc0, p0, sc1, p1 = _top2_ax1(comb, _K * _K)
    iota4 = lax.broadcasted_iota(jnp.int32, (_H, _K * _K, _BLK), 1)
    e0 = jnp.sum(jnp.where(iota4 == p0, cidx, 0), axis=1, keepdims=True)
    e1 = jnp.sum(jnp.where(iota4 == p1, cidx, 0), axis=1, keepdims=True)
    ex = jnp.exp(sc1 - sc0)
    denom = 1.0 + ex
    sw0 = 1.0 / denom
    sw1 = ex / denom

    iota_e = lax.broadcasted_iota(jnp.int32, (_E, _BLK), 0)
    w_acc = jnp.zeros((_E, _BLK), jnp.float32)
    for h in range(_H):
        m0 = iota_e == e0[h]
        m1 = iota_e == e1[h]
        d0 = jnp.sum(jnp.where(m0, dot_all_t, 0.0), axis=0, keepdims=True)
        d1 = jnp.sum(jnp.where(m1, dot_all_t, 0.0), axis=0, keepdims=True)
        act0 = d0 * jax.nn.sigmoid(1.702 * d0) * sw0[h]
        act1 = d1 * jax.nn.sigmoid(1.702 * d1) * sw1[h]
        w_acc = w_acc + jnp.where(m0, act0, 0.0) + jnp.where(m1, act1, 0.0)

    c = lax.dot_general(w_acc, h_all, (((0,), (0,)), ((), ())),
                        preferred_element_type=jnp.float32)
    o_ref[...] = jnp.dot(c, wv_ref[...],
                         preferred_element_type=jnp.float32) * (1.0 / _H)


@functools.partial(jax.jit, static_argnames=())
def _run(xf, wqr, sk1, sk2, el, w1, wu, wv):
    return pl.pallas_call(
        _moe_body,
        grid=(_NBLK,),
        in_specs=[
            pl.BlockSpec((_BLK, _D), lambda i: (i, 0)),
            pl.BlockSpec((_D, 2 * _H * _DHALF), lambda i: (0, 0)),
            pl.BlockSpec((_H * _DHALF, _H * _NSUB), lambda i: (0, 0)),
            pl.BlockSpec((_H * _DHALF, _H * _NSUB), lambda i: (0, 0)),
            pl.BlockSpec((_E, _DL), lambda i: (0, 0)),
            pl.BlockSpec((_DL, _DH), lambda i: (0, 0)),
            pl.BlockSpec((_D, _DH), lambda i: (0, 0)),
            pl.BlockSpec((_DH, _D), lambda i: (0, 0)),
        ],
        out_specs=pl.BlockSpec((_BLK, _D), lambda i: (i, 0)),
        out_shape=jax.ShapeDtypeStruct((_N, _D), jnp.float32),
        compiler_params=pltpu.CompilerParams(
            dimension_semantics=("arbitrary",)),
    )(xf, wqr, sk1, sk2, el, w1, wu, wv)


def kernel(x, expert_latents, W1, W2, Wq, sub_keys):
    xf = x.reshape(_N, _D)
    wqr = Wq.reshape(_D, _H, 2, _DHALF).transpose(0, 2, 1, 3).reshape(
        _D, 2 * _H * _DHALF)
    sk1 = jax.scipy.linalg.block_diag(
        *[sub_keys[0, h].T for h in range(_H)])
    sk2 = jax.scipy.linalg.block_diag(
        *[sub_keys[1, h].T for h in range(_H)])
    wu = W2[:, :_D].T
    wv = W2[:, _D:]
    out = _run(xf, wqr, sk1, sk2, expert_latents, W1, wu, wv)
    return out.reshape(_B, _S, _D)
